# Initial kernel scaffold; baseline (speedup 1.0000x reference)
import jax
import jax.numpy as jnp
from jax.experimental import pallas as pl


def kernel(x, edge_index, edge_vals, alpha, beta, sigma):
    raise NotImplementedError("write your pallas kernel here")


# speedup vs baseline: 5.1101x; 5.1101x over previous
import functools

import jax
import jax.numpy as jnp
from jax import lax
from jax.experimental import pallas as pl
from jax.experimental.pallas import tpu as pltpu
from jax.experimental.pallas import tpu_sc as plsc

N = 10000
D = 128
E = 320000
NUM_ITERS = 5

NC = 2
NS = 16
NW = NC * NS
EPW = E // NW
CHUNK = 400
NCHUNKS = EPW // CHUNK
NPAD = 10240
RPT = NPAD // NS
DH = D // NC
EPS = E // NS

_MESH = dict(core_axis_name="c", subcore_axis_name="s")


def _tc_normalize(x):
    def body(x_ref, xn_ref):
        xv = x_ref[...]
        sq = jnp.sum(xv * xv, axis=1, keepdims=True)
        xn_ref[...] = xv * lax.rsqrt(jnp.maximum(sq, 1e-12))

    return pl.pallas_call(
        body, out_shape=jax.ShapeDtypeStruct((N, D), jnp.float32)
    )(x)


def _sc_edge_weights(xn, rowi, coli, ev, scalin):
    @functools.partial(
        pl.kernel,
        out_type=(
            jax.ShapeDtypeStruct((E,), jnp.float32),
            jax.ShapeDtypeStruct((NC, NPAD), jnp.float32),
        ),
        mesh=plsc.VectorSubcoreMesh(**_MESH),
        compiler_params=pltpu.CompilerParams(needs_layout_passes=False),
        scratch_types=[
            pltpu.VMEM((CHUNK,), jnp.int32),
            pltpu.VMEM((CHUNK,), jnp.int32),
            pltpu.VMEM((CHUNK,), jnp.float32),
            pltpu.VMEM((CHUNK,), jnp.float32),
            pltpu.VMEM((CHUNK, D), jnp.float32),
            pltpu.VMEM((CHUNK, D), jnp.float32),
            pltpu.VMEM((RPT,), jnp.float32),
            pltpu.VMEM((16,), jnp.float32),
            pltpu.VMEM_SHARED((NPAD,), jnp.float32),
            pltpu.SemaphoreType.DMA,
        ],
    )
    def k(xn_h, row_h, col_h, ev_h, scal_h, w_h, np_h,
          ri, ci, evv, wv, bufA, bufB, zbuf, scal_v, nshared, sem):
        cid = lax.axis_index("c")
        sid = lax.axis_index("s")
        wid = sid * NC + cid
        base = wid * EPW

        def z16(g, _):
            zbuf[pl.ds(g * 16, 16)] = jnp.zeros((16,), jnp.float32)
            return 0

        lax.fori_loop(0, RPT // 16, z16, 0)
        pltpu.sync_copy(zbuf, nshared.at[pl.ds(sid * RPT, RPT)])
        plsc.subcore_barrier()

        pltpu.sync_copy(scal_h, scal_v)
        sv = jnp.exp(scal_v[...] * -2.0)
        scale = 0.25 * sv[2]

        def chunk(g, _):
            off = base + g * CHUNK
            pltpu.sync_copy(row_h.at[pl.ds(off, CHUNK)], ri)
            pltpu.sync_copy(col_h.at[pl.ds(off, CHUNK)], ci)
            pltpu.sync_copy(ev_h.at[pl.ds(off, CHUNK)], evv)
            pltpu.async_copy(xn_h.at[ri], bufA, sem).wait()
            pltpu.async_copy(xn_h.at[ci], bufB, sem).wait()

            lane = lax.iota(jnp.int32, 16)

            def group(gg, _):
                sl = pl.ds(gg * 16, 16)
                dvec = jnp.zeros((16,), jnp.float32)
                for k in range(16):
                    e = gg * 16 + k
                    p = [bufA[e, pl.ds(s * 16, 16)] * bufB[e, pl.ds(s * 16, 16)]
                         for s in range(8)]
                    acc = (((p[0] + p[1]) + (p[2] + p[3]))
                           + ((p[4] + p[5]) + (p[6] + p[7])))
                    dsum = jnp.sum(acc)
                    dvec = jnp.where(lane == k, dsum, dvec)
                rvec = ri[sl]
                cvec = ci[sl]
                dotv = jnp.where(rvec == cvec, 0.0, dvec)
                wv[sl] = evv[sl] * jnp.exp(dotv * scale)
                return 0

            lax.fori_loop(0, CHUNK // 16, group, 0)
            pltpu.sync_copy(wv, w_h.at[pl.ds(off, CHUNK)])
            pltpu.sync_copy(wv, nshared.at[ri], add=True)
            return 0

        lax.fori_loop(0, NCHUNKS, chunk, 0)
        plsc.subcore_barrier()
        pltpu.sync_copy(nshared.at[pl.ds(sid * RPT, RPT)],
                        np_h.at[cid, pl.ds(sid * RPT, RPT)])

    return k(xn, rowi, coli, ev, scalin)


def _sc_spmm(out2, rowi, coli, w):
    @functools.partial(
        pl.kernel,
        out_type=jax.ShapeDtypeStruct((NC, NPAD, DH), jnp.float32),
        mesh=plsc.VectorSubcoreMesh(**_MESH),
        compiler_params=pltpu.CompilerParams(
            needs_layout_passes=False, use_tc_tiling_on_sc=False),
        scratch_types=[
            pltpu.VMEM((CHUNK,), jnp.int32),
            pltpu.VMEM((CHUNK,), jnp.int32),
            pltpu.VMEM((CHUNK,), jnp.float32),
            pltpu.VMEM((CHUNK, DH), jnp.float32),
            pltpu.VMEM((CHUNK, DH), jnp.float32),
            pltpu.VMEM_SHARED((NPAD, DH), jnp.float32),
            pltpu.SemaphoreType.DMA,
        ],
    )
    def k(out_h, row_h, col_h, w_h, ap_h, ri, ci, wv, buf, buf2, acc, sem):
        cid = lax.axis_index("c")
        sid = lax.axis_index("s")
        base = sid * EPS

        def zb(i, _):
            for sreg in range(DH // 16):
                buf2[i, pl.ds(sreg * 16, 16)] = jnp.zeros((16,), jnp.float32)
            return 0

        lax.fori_loop(0, CHUNK, zb, 0)
        r0 = sid * RPT
        pltpu.sync_copy(buf2.at[pl.ds(0, CHUNK)], acc.at[pl.ds(r0, CHUNK)])
        pltpu.sync_copy(buf2.at[pl.ds(0, RPT - CHUNK)],
                        acc.at[pl.ds(r0 + CHUNK, RPT - CHUNK)])
        plsc.subcore_barrier()

        def chunk(g, _):
            off = base + g * CHUNK
            pltpu.sync_copy(row_h.at[pl.ds(off, CHUNK)], ri)
            pltpu.sync_copy(col_h.at[pl.ds(off, CHUNK)], ci)
            pltpu.sync_copy(w_h.at[pl.ds(off, CHUNK)], wv)

            def cix(gg, _):
                sl = pl.ds(gg * 16, 16)
                ci[sl] = ci[sl] * 2 + cid
                return 0

            lax.fori_loop(0, CHUNK // 16, cix, 0)
            pltpu.async_copy(out_h.at[ci], buf, sem).wait()

            def group(gg, _):
                sl = pl.ds(gg * 16, 16)
                wgrp = wv[sl]
                for k in range(16):
                    e = gg * 16 + k
                    we = wgrp[k]
                    for sreg in range(DH // 16):
                        s2 = pl.ds(sreg * 16, 16)
                        buf2[e, s2] = buf[e, s2] * we
                return 0

            lax.fori_loop(0, CHUNK // 16, group, 0)
            pltpu.sync_copy(buf2, acc.at[ri], add=True)
            return 0

        lax.fori_loop(0, EPS // CHUNK, chunk, 0)
        plsc.subcore_barrier()
        pltpu.sync_copy(acc.at[pl.ds(r0, RPT)], ap_h.at[cid, pl.ds(r0, RPT)])

    return k(out2, rowi, coli, w)


def _tc_update(x, out, ap, npart, alpha2, beta2):
    BLK = 1280

    def body(x_ref, o_ref, ap_ref, np_ref, a_ref, b_ref, on_ref):
        ae = jnp.exp(a_ref[0, 0])
        be = jnp.exp(b_ref[0, 0])
        xv = x_ref[...]
        ov = o_ref[...]
        apm = ap_ref[...]
        apv = jnp.concatenate([apm[0], apm[1]], axis=1)
        nv = jnp.sum(np_ref[...], axis=0)
        denom = be + nv[:, None] * ae + ae
        on_ref[...] = (xv * be + (apv + ov) * ae) / denom

    return pl.pallas_call(
        body,
        grid=(NPAD // BLK,),
        in_specs=[
            pl.BlockSpec((BLK, D), lambda i: (i, 0)),
            pl.BlockSpec((BLK, D), lambda i: (i, 0)),
            pl.BlockSpec((NC, BLK, DH), lambda i: (0, i, 0)),
            pl.BlockSpec((NC, BLK), lambda i: (0, i)),
            pl.BlockSpec((1, 1), lambda i: (0, 0)),
            pl.BlockSpec((1, 1), lambda i: (0, 0)),
        ],
        out_specs=pl.BlockSpec((BLK, D), lambda i: (i, 0)),
        out_shape=jax.ShapeDtypeStruct((N, D), jnp.float32),
    )(x, out, ap, npart, alpha2, beta2)


def kernel(x, edge_index, edge_vals, alpha, beta, sigma):
    ei = edge_index.astype(jnp.int32)
    rowi = ei[0]
    coli = ei[1]
    scalin = jnp.concatenate(
        [alpha.astype(jnp.float32), beta.astype(jnp.float32),
         sigma.astype(jnp.float32), jnp.zeros((13,), jnp.float32)])
    alpha2 = alpha.astype(jnp.float32).reshape(1, 1)
    beta2 = beta.astype(jnp.float32).reshape(1, 1)

    xn = _tc_normalize(x)
    w, npart = _sc_edge_weights(xn, rowi, coli, edge_vals, scalin)
    out = x
    for _ in range(NUM_ITERS):
        ap = _sc_spmm(out.reshape(NC * N, DH), rowi, coli, w)
        out = _tc_update(x, out, ap, npart, alpha2, beta2)
    return out
